# Initial kernel scaffold; baseline (speedup 1.0000x reference)
import jax
import jax.numpy as jnp
from jax.experimental import pallas as pl


def kernel(x, wc, bc, w2t, b2t):
    raise NotImplementedError("write your pallas kernel here")


# speedup vs baseline: 4.2647x; 4.2647x over previous
import jax
import jax.numpy as jnp
from jax.experimental import pallas as pl
from jax.experimental.pallas import tpu as pltpu

_LANE = 128
_W = 8
_CO = 8
_CI = 3


def _fused_kernel(x_ref, wc_ref, bc_ref, w2t_ref, b2_ref, o_ref):
    hwp = x_ref.shape[1] // _CI
    x0 = x_ref[:, 0 * hwp:1 * hwp]
    x1 = x_ref[:, 1 * hwp:2 * hwp]
    x2 = x_ref[:, 2 * hwp:3 * hwp]
    w2t = w2t_ref[...]
    b2 = b2_ref[...]
    for co in range(_CO):
        v = jnp.maximum(
            x0 * wc_ref[co, 0] + x1 * wc_ref[co, 1] + x2 * wc_ref[co, 2]
            + bc_ref[co, 0], 0.0)
        base = co * hwp
        for k in range(hwp // _LANE):
            lo = k * _LANE
            y = jnp.dot(v[:, lo:lo + _LANE], w2t,
                        preferred_element_type=jnp.float32)
            o_ref[:, base + lo:base + lo + _LANE] = y + b2


def _pick_bn(n):
    groups = pl.cdiv(n, 8)
    if groups <= 1:
        return 8
    return 8 * min(8, max(1, groups // 2))


def kernel(x, wc, bc, w2t, b2t):
    N, C, H, W = x.shape
    HW = H * W
    HWp = pl.cdiv(HW, _LANE) * _LANE
    bn = _pick_bn(N)
    Np = pl.cdiv(N, bn) * bn

    x_flat = x.reshape(N, C, HW)
    if HWp != HW:
        x_flat = jnp.pad(x_flat, ((0, 0), (0, 0), (0, HWp - HW)))
    if Np != N:
        x_flat = jnp.pad(x_flat, ((0, Np - N), (0, 0), (0, 0)))
    x2d = x_flat.reshape(Np, C * HWp)

    out2d = pl.pallas_call(
        _fused_kernel,
        out_shape=jax.ShapeDtypeStruct((Np, _CO * HWp), jnp.float32),
        grid=(Np // bn,),
        in_specs=[
            pl.BlockSpec((bn, C * HWp), lambda n: (n, 0)),
            pl.BlockSpec((_CO, _CI), lambda n: (0, 0)),
            pl.BlockSpec((_CO, 1), lambda n: (0, 0)),
            pl.BlockSpec((_LANE, _LANE), lambda n: (0, 0)),
            pl.BlockSpec((1, _LANE), lambda n: (0, 0)),
        ],
        out_specs=pl.BlockSpec((bn, _CO * HWp), lambda n: (n, 0)),
        compiler_params=pltpu.CompilerParams(
            dimension_semantics=("parallel",)),
    )(x2d, wc, bc, w2t, b2t)

    return out2d.reshape(Np, _CO, HWp)[:N, :, :HW].reshape(N, _CO, H, W)


# speedup vs baseline: 7.3462x; 1.7225x over previous
import jax
import jax.numpy as jnp
from jax.experimental import pallas as pl
from jax.experimental.pallas import tpu as pltpu

_LANE = 128
_W = 8
_CO = 8
_CI = 3


def _fused_kernel(x_ref, wc_ref, bc_ref, w2b_ref, b2b_ref, o_ref):
    x0 = x_ref[:, 0 * _W:1 * _W, :]
    x1 = x_ref[:, 1 * _W:2 * _W, :]
    x2 = x_ref[:, 2 * _W:3 * _W, :]
    b2b = b2b_ref[...]
    for co in range(_CO):
        v = jnp.maximum(
            x0 * wc_ref[co, 0] + x1 * wc_ref[co, 1] + x2 * wc_ref[co, 2]
            + bc_ref[co, 0], 0.0)
        y = b2b + v[:, 0:1, :] * w2b_ref[0]
        for jp in range(1, _W):
            y = y + v[:, jp:jp + 1, :] * w2b_ref[jp]
        o_ref[:, co * _W:(co + 1) * _W, :] = y


def _pick_bn(n):
    if n <= 16:
        return max(n // 2, 1)
    return min(64, n // 2)


def kernel(x, wc, bc, w2t, b2t):
    N, C, H, W = x.shape
    Hp = pl.cdiv(H, _LANE) * _LANE
    bn = _pick_bn(N)
    Np = pl.cdiv(N, bn) * bn

    x3 = jnp.transpose(x, (0, 1, 3, 2))
    if Hp != H:
        x3 = jnp.pad(x3, ((0, 0), (0, 0), (0, 0), (0, Hp - H)))
    if Np != N:
        x3 = jnp.pad(x3, ((0, Np - N), (0, 0), (0, 0), (0, 0)))
    x3 = x3.reshape(Np, C * W, Hp)

    w2sub = w2t[:_W, :_W]
    w2b = jnp.broadcast_to(w2sub[:, :, None], (_W, _W, Hp))
    b2b = jnp.broadcast_to(b2t[0, :_W][:, None], (_W, Hp))

    out3 = pl.pallas_call(
        _fused_kernel,
        out_shape=jax.ShapeDtypeStruct((Np, _CO * _W, Hp), jnp.float32),
        grid=(Np // bn,),
        in_specs=[
            pl.BlockSpec((bn, C * W, Hp), lambda n: (n, 0, 0)),
            pl.BlockSpec((_CO, _CI), lambda n: (0, 0)),
            pl.BlockSpec((_CO, 1), lambda n: (0, 0)),
            pl.BlockSpec((_W, _W, Hp), lambda n: (0, 0, 0)),
            pl.BlockSpec((_W, Hp), lambda n: (0, 0)),
        ],
        out_specs=pl.BlockSpec((bn, _CO * _W, Hp), lambda n: (n, 0, 0)),
        compiler_params=pltpu.CompilerParams(
            dimension_semantics=("parallel",)),
    )(x3, wc, bc, w2b, b2b)

    out4 = out3.reshape(Np, _CO, _W, Hp)[:N, :, :, :H]
    return jnp.transpose(out4, (0, 1, 3, 2))


# speedup vs baseline: 13.2607x; 1.8051x over previous
import jax
import jax.numpy as jnp
from jax.experimental import pallas as pl
from jax.experimental.pallas import tpu as pltpu

_LANE = 128
_W = 8
_CO = 8
_CI = 3


def _fused_kernel(x_ref, wc_ref, bc_ref, w2k_ref, b2k_ref, o_ref):
    bn = x_ref.shape[0]
    x0 = x_ref[:, 0 * _W:1 * _W, :]
    x1 = x_ref[:, 1 * _W:2 * _W, :]
    x2 = x_ref[:, 2 * _W:3 * _W, :]
    v = jnp.concatenate(
        [jnp.maximum(
            x0 * wc_ref[co, 0] + x1 * wc_ref[co, 1] + x2 * wc_ref[co, 2]
            + bc_ref[co, 0], 0.0)
         for co in range(_CO)], axis=1)
    w2k = w2k_ref[...]
    b2k = b2k_ref[...]
    for i in range(bn):
        o_ref[i] = jnp.dot(w2k, v[i],
                           preferred_element_type=jnp.float32) + b2k


def _pick_bn(n):
    if n <= 16:
        return max(n // 2, 1)
    return min(32, n // 2)


def kernel(x, wc, bc, w2t, b2t):
    N, C, H, W = x.shape
    Hp = pl.cdiv(H, _LANE) * _LANE
    bn = _pick_bn(N)
    Np = pl.cdiv(N, bn) * bn

    x3 = jnp.transpose(x, (0, 1, 3, 2))
    if Hp != H:
        x3 = jnp.pad(x3, ((0, 0), (0, 0), (0, 0), (0, Hp - H)))
    if Np != N:
        x3 = jnp.pad(x3, ((0, Np - N), (0, 0), (0, 0), (0, 0)))
    x3 = x3.reshape(Np, C * W, Hp)

    w2k = jnp.kron(jnp.eye(_CO, dtype=jnp.float32), w2t[:_W, :_W].T)
    b2k = jnp.broadcast_to(
        jnp.tile(b2t[0, :_W], _CO)[:, None], (_CO * _W, Hp))

    out3 = pl.pallas_call(
        _fused_kernel,
        out_shape=jax.ShapeDtypeStruct((Np, _CO * _W, Hp), jnp.float32),
        grid=(Np // bn,),
        in_specs=[
            pl.BlockSpec((bn, C * W, Hp), lambda n: (n, 0, 0)),
            pl.BlockSpec((_CO, _CI), lambda n: (0, 0)),
            pl.BlockSpec((_CO, 1), lambda n: (0, 0)),
            pl.BlockSpec((_CO * _W, _CO * _W), lambda n: (0, 0)),
            pl.BlockSpec((_CO * _W, Hp), lambda n: (0, 0)),
        ],
        out_specs=pl.BlockSpec((bn, _CO * _W, Hp), lambda n: (n, 0, 0)),
        compiler_params=pltpu.CompilerParams(
            dimension_semantics=("parallel",)),
    )(x3, wc, bc, w2k, b2k)

    out4 = out3.reshape(Np, _CO, _W, Hp)[:N, :, :, :H]
    return jnp.transpose(out4, (0, 1, 3, 2))


# speedup vs baseline: 17.8484x; 1.3460x over previous
import jax
import jax.numpy as jnp
from jax.experimental import pallas as pl
from jax.experimental.pallas import tpu as pltpu

_LANE = 128
_W = 8
_CO = 8
_CI = 3


def _fused_kernel(x_ref, wc_ref, bc_ref, w2k_ref, b2k_ref, o_ref):
    bn = x_ref.shape[0]
    x0 = x_ref[:, 0 * _W:1 * _W, :]
    x1 = x_ref[:, 1 * _W:2 * _W, :]
    x2 = x_ref[:, 2 * _W:3 * _W, :]
    v = jnp.concatenate(
        [jnp.maximum(
            x0 * wc_ref[co, 0] + x1 * wc_ref[co, 1] + x2 * wc_ref[co, 2]
            + bc_ref[co, 0], 0.0)
         for co in range(_CO)], axis=1)
    w2k = w2k_ref[...]
    b2k = b2k_ref[...]
    for i in range(bn):
        o_ref[i] = jnp.dot(w2k, v[i],
                           preferred_element_type=jnp.float32) + b2k


def _pick_bn(n):
    if n <= 16:
        return max(n // 2, 1)
    return min(64, n // 2)


def kernel(x, wc, bc, w2t, b2t):
    N, C, H, W = x.shape
    Hp = pl.cdiv(H, _LANE) * _LANE
    bn = _pick_bn(N)
    Np = pl.cdiv(N, bn) * bn

    x3 = jnp.transpose(x, (0, 1, 3, 2))
    if Hp != H:
        x3 = jnp.pad(x3, ((0, 0), (0, 0), (0, 0), (0, Hp - H)))
    if Np != N:
        x3 = jnp.pad(x3, ((0, Np - N), (0, 0), (0, 0), (0, 0)))
    x3 = x3.reshape(Np, C * W, Hp)

    w2k = jnp.kron(jnp.eye(_CO, dtype=jnp.float32), w2t[:_W, :_W].T)
    b2k = jnp.broadcast_to(
        jnp.tile(b2t[0, :_W], _CO)[:, None], (_CO * _W, Hp))

    out3 = pl.pallas_call(
        _fused_kernel,
        out_shape=jax.ShapeDtypeStruct((Np, _CO * _W, Hp), jnp.float32),
        grid=(Np // bn,),
        in_specs=[
            pl.BlockSpec((bn, C * W, Hp), lambda n: (n, 0, 0)),
            pl.BlockSpec((_CO, _CI), lambda n: (0, 0)),
            pl.BlockSpec((_CO, 1), lambda n: (0, 0)),
            pl.BlockSpec((_CO * _W, _CO * _W), lambda n: (0, 0)),
            pl.BlockSpec((_CO * _W, Hp), lambda n: (0, 0)),
        ],
        out_specs=pl.BlockSpec((bn, _CO * _W, Hp), lambda n: (n, 0, 0)),
        compiler_params=pltpu.CompilerParams(
            dimension_semantics=("parallel",)),
    )(x3, wc, bc, w2k, b2k)

    out4 = out3.reshape(Np, _CO, _W, Hp)[:N, :, :, :H]
    return jnp.transpose(out4, (0, 1, 3, 2))


# speedup vs baseline: 21.6424x; 1.2126x over previous
import jax
import jax.numpy as jnp
from jax.experimental import pallas as pl
from jax.experimental.pallas import tpu as pltpu

_LANE = 128
_W = 8
_CO = 8
_CI = 3


def _fused_kernel(x_ref, wc_ref, bc_ref, w2k_ref, b2k_ref, o_ref):
    bn = x_ref.shape[0]
    x0 = x_ref[:, 0 * _W:1 * _W, :]
    x1 = x_ref[:, 1 * _W:2 * _W, :]
    x2 = x_ref[:, 2 * _W:3 * _W, :]
    v = jnp.concatenate(
        [jnp.maximum(
            x0 * wc_ref[co, 0] + x1 * wc_ref[co, 1] + x2 * wc_ref[co, 2]
            + bc_ref[co, 0], 0.0)
         for co in range(_CO)], axis=1)
    w2k = w2k_ref[...]
    b2k = b2k_ref[...]
    for i in range(bn):
        o_ref[i] = jnp.dot(w2k, v[i],
                           preferred_element_type=jnp.float32) + b2k


def _pick_bn(n):
    if n <= 16:
        return max(n // 2, 1)
    return min(128, n // 2)


def kernel(x, wc, bc, w2t, b2t):
    N, C, H, W = x.shape
    Hp = pl.cdiv(H, _LANE) * _LANE
    bn = _pick_bn(N)
    Np = pl.cdiv(N, bn) * bn

    x3 = jnp.transpose(x, (0, 1, 3, 2))
    if Hp != H:
        x3 = jnp.pad(x3, ((0, 0), (0, 0), (0, 0), (0, Hp - H)))
    if Np != N:
        x3 = jnp.pad(x3, ((0, Np - N), (0, 0), (0, 0), (0, 0)))
    x3 = x3.reshape(Np, C * W, Hp)

    w2k = jnp.kron(jnp.eye(_CO, dtype=jnp.float32), w2t[:_W, :_W].T)
    b2k = jnp.broadcast_to(
        jnp.tile(b2t[0, :_W], _CO)[:, None], (_CO * _W, Hp))

    out3 = pl.pallas_call(
        _fused_kernel,
        out_shape=jax.ShapeDtypeStruct((Np, _CO * _W, Hp), jnp.float32),
        grid=(Np // bn,),
        in_specs=[
            pl.BlockSpec((bn, C * W, Hp), lambda n: (n, 0, 0)),
            pl.BlockSpec((_CO, _CI), lambda n: (0, 0)),
            pl.BlockSpec((_CO, 1), lambda n: (0, 0)),
            pl.BlockSpec((_CO * _W, _CO * _W), lambda n: (0, 0)),
            pl.BlockSpec((_CO * _W, Hp), lambda n: (0, 0)),
        ],
        out_specs=pl.BlockSpec((bn, _CO * _W, Hp), lambda n: (n, 0, 0)),
        compiler_params=pltpu.CompilerParams(
            dimension_semantics=("parallel",)),
    )(x3, wc, bc, w2k, b2k)

    out4 = out3.reshape(Np, _CO, _W, Hp)[:N, :, :, :H]
    return jnp.transpose(out4, (0, 1, 3, 2))


# speedup vs baseline: 23.8842x; 1.1036x over previous
import jax
import jax.numpy as jnp
from jax.experimental import pallas as pl
from jax.experimental.pallas import tpu as pltpu

_LANE = 128
_W = 8
_CO = 8
_CI = 3


def _fused_kernel(x_ref, wc_ref, bc_ref, w2k_ref, b2k_ref, o_ref):
    bn = x_ref.shape[0]
    x0 = x_ref[:, 0 * _W:1 * _W, :]
    x1 = x_ref[:, 1 * _W:2 * _W, :]
    x2 = x_ref[:, 2 * _W:3 * _W, :]
    v = jnp.concatenate(
        [jnp.maximum(
            x0 * wc_ref[co, 0] + x1 * wc_ref[co, 1] + x2 * wc_ref[co, 2]
            + bc_ref[co, 0], 0.0)
         for co in range(_CO)], axis=1)
    w2k = w2k_ref[...]
    b2k = b2k_ref[...]
    for i in range(bn):
        o_ref[i] = jnp.dot(w2k, v[i],
                           preferred_element_type=jnp.float32) + b2k


def _pick_bn(n):
    if n <= 16:
        return max(n // 2, 1)
    return min(256, n // 2)


def kernel(x, wc, bc, w2t, b2t):
    N, C, H, W = x.shape
    Hp = pl.cdiv(H, _LANE) * _LANE
    bn = _pick_bn(N)
    Np = pl.cdiv(N, bn) * bn

    x3 = jnp.transpose(x, (0, 1, 3, 2))
    if Hp != H:
        x3 = jnp.pad(x3, ((0, 0), (0, 0), (0, 0), (0, Hp - H)))
    if Np != N:
        x3 = jnp.pad(x3, ((0, Np - N), (0, 0), (0, 0), (0, 0)))
    x3 = x3.reshape(Np, C * W, Hp)

    w2k = jnp.kron(jnp.eye(_CO, dtype=jnp.float32), w2t[:_W, :_W].T)
    b2k = jnp.broadcast_to(
        jnp.tile(b2t[0, :_W], _CO)[:, None], (_CO * _W, Hp))

    out3 = pl.pallas_call(
        _fused_kernel,
        out_shape=jax.ShapeDtypeStruct((Np, _CO * _W, Hp), jnp.float32),
        grid=(Np // bn,),
        in_specs=[
            pl.BlockSpec((bn, C * W, Hp), lambda n: (n, 0, 0)),
            pl.BlockSpec((_CO, _CI), lambda n: (0, 0)),
            pl.BlockSpec((_CO, 1), lambda n: (0, 0)),
            pl.BlockSpec((_CO * _W, _CO * _W), lambda n: (0, 0)),
            pl.BlockSpec((_CO * _W, Hp), lambda n: (0, 0)),
        ],
        out_specs=pl.BlockSpec((bn, _CO * _W, Hp), lambda n: (n, 0, 0)),
        compiler_params=pltpu.CompilerParams(
            dimension_semantics=("parallel",)),
    )(x3, wc, bc, w2k, b2k)

    out4 = out3.reshape(Np, _CO, _W, Hp)[:N, :, :, :H]
    return jnp.transpose(out4, (0, 1, 3, 2))
